# Initial kernel scaffold; baseline (speedup 1.0000x reference)
import jax
import jax.numpy as jnp
from jax.experimental import pallas as pl


def kernel(x, edge_index, identifiers, degrees, edge_features, W1, b1, W2, b2):
    raise NotImplementedError("write your pallas kernel here")


# speedup vs baseline: 3.9450x; 3.9450x over previous
import jax
import jax.numpy as jnp
from jax import lax
from jax.experimental import pallas as pl
from jax.experimental.pallas import tpu as pltpu
from jax.experimental.pallas import tpu_sc as plsc

N = 10000
E = 320000
D = 128

NC = 2
NS = 16
NW = NC * NS
EPW = E // NW
C = 80
G = EPW // C
NPAD = 10240
RPT = NPAD // NS
LPR = D // 16
IPC = C // 16


def _sc_edge_body(src_hbm, dst_hbm, x_hbm, ef_hbm, zeros_hbm, out_hbm,
                  srcv, dstv, sidx, didx, efv, xv, agg_sh):
    c = lax.axis_index("c")
    s = lax.axis_index("s")
    wid = c * NS + s

    pltpu.sync_copy(zeros_hbm, agg_sh.at[pl.ds(s * RPT, RPT)])

    ibase = wid * EPW
    pltpu.sync_copy(src_hbm.at[pl.ds(ibase, EPW)], srcv)
    pltpu.sync_copy(dst_hbm.at[pl.ds(ibase, EPW)], dstv)

    plsc.subcore_barrier()

    def chunk(g, carry):
        ebase = pl.multiple_of(wid * EPW + g * C, 8)
        pltpu.sync_copy(ef_hbm.at[pl.ds(ebase, C)], efv)
        for j in range(IPC):
            sl = pl.ds(j * 16, 16)
            src_sl = pl.ds(g * C + j * 16, 16)
            sidx[sl] = srcv[src_sl]
            didx[sl] = dstv[src_sl]
        pltpu.sync_copy(x_hbm.at[sidx], xv)

        def row(r, carry2):
            for j in range(LPR):
                sl = pl.ds(j * 16, 16)
                xv[r, sl] = jnp.maximum(xv[r, sl] + efv[r, sl], 0.0)
            return carry2

        lax.fori_loop(0, C, row, 0)

        pltpu.sync_copy(xv, agg_sh.at[didx], add=True)
        return carry

    lax.fori_loop(0, G, chunk, 0)

    plsc.subcore_barrier()

    rbase = s * RPT
    pltpu.sync_copy(agg_sh.at[pl.ds(rbase, RPT)],
                    out_hbm.at[c, pl.ds(rbase, RPT)])


@jax.jit
def _sc_edge_phase(src, dst, x, ef, zeros):
    mesh = plsc.VectorSubcoreMesh(core_axis_name="c", subcore_axis_name="s")
    k = pl.kernel(
        _sc_edge_body,
        out_type=jax.ShapeDtypeStruct((NC, NPAD, D), jnp.float32),
        mesh=mesh,
        compiler_params=pltpu.CompilerParams(use_tc_tiling_on_sc=False),
        scratch_types=[
            pltpu.VMEM((EPW,), jnp.int32),
            pltpu.VMEM((EPW,), jnp.int32),
            pltpu.VMEM((C,), jnp.int32),
            pltpu.VMEM((C,), jnp.int32),
            pltpu.VMEM((C, D), jnp.float32),
            pltpu.VMEM((C, D), jnp.float32),
            pltpu.VMEM_SHARED((NPAD, D), jnp.float32),
        ],
    )
    return k(src, dst, x, ef, zeros)


def _mlp_body(x_ref, p_ref, w1_ref, b1_ref, w2_ref, b2_ref, o_ref):
    h = x_ref[...] + p_ref[0] + p_ref[1]
    h = jnp.maximum(
        jnp.dot(h, w1_ref[...], preferred_element_type=jnp.float32) + b1_ref[...],
        0.0)
    o_ref[...] = (
        jnp.dot(h, w2_ref[...], preferred_element_type=jnp.float32) + b2_ref[...])


BR = 400


@jax.jit
def _mlp_phase(x, p, W1, b1, W2, b2):
    grid = (N // BR,)
    return pl.pallas_call(
        _mlp_body,
        grid=grid,
        in_specs=[
            pl.BlockSpec((BR, D), lambda i: (i, 0)),
            pl.BlockSpec((NC, BR, D), lambda i: (0, i, 0)),
            pl.BlockSpec((D, D), lambda i: (0, 0)),
            pl.BlockSpec((1, D), lambda i: (0, 0)),
            pl.BlockSpec((D, D), lambda i: (0, 0)),
            pl.BlockSpec((1, D), lambda i: (0, 0)),
        ],
        out_specs=pl.BlockSpec((BR, D), lambda i: (i, 0)),
        out_shape=jax.ShapeDtypeStruct((N, D), jnp.float32),
    )(x, p, W1, b1.reshape(1, D), W2, b2.reshape(1, D))


def kernel(x, edge_index, identifiers, degrees, edge_features, W1, b1, W2, b2):
    src = edge_index[0]
    dst = edge_index[1]
    zeros = jnp.zeros((RPT, D), jnp.float32)
    p = _sc_edge_phase(src, dst, x, edge_features, zeros)
    return _mlp_phase(x, p[:, :N], W1, b1, W2, b2)


# speedup vs baseline: 5.9992x; 1.5207x over previous
import jax
import jax.numpy as jnp
from jax import lax
from jax.experimental import pallas as pl
from jax.experimental.pallas import tpu as pltpu
from jax.experimental.pallas import tpu_sc as plsc

N = 10000
E = 320000
D = 128

NC = 2
NS = 16
NW = NC * NS
EPW = E // NW
C = 40
G = EPW // C
NPAD = 10240
RPT = NPAD // NS
LPR = D // 16


def _sc_edge_body(src_hbm, dst_hbm, x_hbm, ef_hbm, zeros_hbm, out_hbm,
                  sidx0, sidx1, didx0, didx1,
                  efv0, efv1, xv0, xv1, mv0, mv1,
                  sef0, sef1, sx0, sx1, ssc0, ssc1, ssi0, ssi1, ssd0, ssd1,
                  agg_sh):
    sidx = (sidx0, sidx1)
    didx = (didx0, didx1)
    efv = (efv0, efv1)
    xv = (xv0, xv1)
    mv = (mv0, mv1)
    sef = (sef0, sef1)
    sx = (sx0, sx1)
    ssc = (ssc0, ssc1)
    ssi = (ssi0, ssi1)
    ssd = (ssd0, ssd1)

    c = lax.axis_index("c")
    s = lax.axis_index("s")
    wid = c * NS + s
    ibase = wid * EPW

    pltpu.sync_copy(zeros_hbm, agg_sh.at[pl.ds(s * RPT, RPT)])

    plsc.subcore_barrier()

    def eb(g):
        return pl.multiple_of(ibase + g * C, 8)

    def start_sidx(g, b):
        pltpu.async_copy(src_hbm.at[pl.ds(eb(g), C)], sidx[b], ssi[b])

    def wait_sidx(b):
        pltpu.make_async_copy(src_hbm.at[pl.ds(0, C)], sidx[b], ssi[b]).wait()

    def start_didx(g, b):
        pltpu.async_copy(dst_hbm.at[pl.ds(eb(g), C)], didx[b], ssd[b])

    def wait_didx(b):
        pltpu.make_async_copy(dst_hbm.at[pl.ds(0, C)], didx[b], ssd[b]).wait()

    def start_in(g, b):
        pltpu.async_copy(ef_hbm.at[pl.ds(eb(g), C)], efv[b], sef[b])
        pltpu.async_copy(x_hbm.at[sidx[b]], xv[b], sx[b])

    def wait_in(b):
        pltpu.make_async_copy(ef_hbm.at[pl.ds(0, C)], efv[b], sef[b]).wait()
        pltpu.make_async_copy(x_hbm.at[sidx[b]], xv[b], sx[b]).wait()

    def compute(b):
        def row(r, carry):
            for j in range(LPR):
                sl = pl.ds(j * 16, 16)
                mv[b][r, sl] = jnp.maximum(xv[b][r, sl] + efv[b][r, sl], 0.0)
            return carry
        lax.fori_loop(0, C, row, 0)

    def start_scatter(b):
        pltpu.async_copy(mv[b], agg_sh.at[didx[b]], ssc[b], add=True)

    def wait_scatter(b):
        pltpu.make_async_copy(mv[b], agg_sh.at[didx[b]], ssc[b]).wait()

    def iteration(g, b, first):
        def start_next():
            wait_sidx(b ^ 1)
            start_in(g + 1, b ^ 1)

        def start_next_idx():
            start_sidx(g + 2, b)

        if first:
            if g + 1 < G:
                start_next()
        else:
            @pl.when(g + 1 < G)
            def _():
                start_next()
        wait_in(b)
        if not first:
            wait_scatter(b)
        start_didx(g, b)
        compute(b)
        wait_didx(b)
        start_scatter(b)
        if first:
            if g + 2 < G:
                start_next_idx()
        else:
            @pl.when(g + 2 < G)
            def _():
                start_next_idx()

    start_sidx(0, 0)
    wait_sidx(0)
    start_in(0, 0)
    start_sidx(1, 1)

    iteration(0, 0, True)
    iteration(1, 1, True)

    def pair(i, carry):
        g = 2 + 2 * i
        iteration(g, 0, False)
        iteration(g + 1, 1, False)
        return carry

    lax.fori_loop(0, (G - 2) // 2, pair, 0)

    wait_scatter(0)
    wait_scatter(1)

    plsc.subcore_barrier()

    rbase = s * RPT
    pltpu.sync_copy(agg_sh.at[pl.ds(rbase, RPT)],
                    out_hbm.at[c, pl.ds(rbase, RPT)])


@jax.jit
def _sc_edge_phase(src, dst, x, ef, zeros):
    mesh = plsc.VectorSubcoreMesh(core_axis_name="c", subcore_axis_name="s")
    k = pl.kernel(
        _sc_edge_body,
        out_type=jax.ShapeDtypeStruct((NC, NPAD, D), jnp.float32),
        mesh=mesh,
        compiler_params=pltpu.CompilerParams(use_tc_tiling_on_sc=False),
        scratch_types=[
            pltpu.VMEM((C,), jnp.int32),
            pltpu.VMEM((C,), jnp.int32),
            pltpu.VMEM((C,), jnp.int32),
            pltpu.VMEM((C,), jnp.int32),
            pltpu.VMEM((C, D), jnp.float32),
            pltpu.VMEM((C, D), jnp.float32),
            pltpu.VMEM((C, D), jnp.float32),
            pltpu.VMEM((C, D), jnp.float32),
            pltpu.VMEM((C, D), jnp.float32),
            pltpu.VMEM((C, D), jnp.float32),
            pltpu.SemaphoreType.DMA,
            pltpu.SemaphoreType.DMA,
            pltpu.SemaphoreType.DMA,
            pltpu.SemaphoreType.DMA,
            pltpu.SemaphoreType.DMA,
            pltpu.SemaphoreType.DMA,
            pltpu.SemaphoreType.DMA,
            pltpu.SemaphoreType.DMA,
            pltpu.SemaphoreType.DMA,
            pltpu.SemaphoreType.DMA,
            pltpu.VMEM_SHARED((NPAD, D), jnp.float32),
        ],
    )
    return k(src, dst, x, ef, zeros)


def _mlp_body(x_ref, p_ref, w1_ref, b1_ref, w2_ref, b2_ref, o_ref):
    h = x_ref[...] + p_ref[0] + p_ref[1]
    h = jnp.maximum(
        jnp.dot(h, w1_ref[...], preferred_element_type=jnp.float32) + b1_ref[...],
        0.0)
    o_ref[...] = (
        jnp.dot(h, w2_ref[...], preferred_element_type=jnp.float32) + b2_ref[...])


BR = 400


@jax.jit
def _mlp_phase(x, p, W1, b1, W2, b2):
    grid = (N // BR,)
    return pl.pallas_call(
        _mlp_body,
        grid=grid,
        in_specs=[
            pl.BlockSpec((BR, D), lambda i: (i, 0)),
            pl.BlockSpec((NC, BR, D), lambda i: (0, i, 0)),
            pl.BlockSpec((D, D), lambda i: (0, 0)),
            pl.BlockSpec((1, D), lambda i: (0, 0)),
            pl.BlockSpec((D, D), lambda i: (0, 0)),
            pl.BlockSpec((1, D), lambda i: (0, 0)),
        ],
        out_specs=pl.BlockSpec((BR, D), lambda i: (i, 0)),
        out_shape=jax.ShapeDtypeStruct((N, D), jnp.float32),
    )(x, p, W1, b1.reshape(1, D), W2, b2.reshape(1, D))


def kernel(x, edge_index, identifiers, degrees, edge_features, W1, b1, W2, b2):
    src = edge_index[0]
    dst = edge_index[1]
    zeros = jnp.zeros((RPT, D), jnp.float32)
    p = _sc_edge_phase(src, dst, x, edge_features, zeros)
    return _mlp_phase(x, p[:, :N], W1, b1, W2, b2)


# speedup vs baseline: 6.5907x; 1.0986x over previous
import jax
import jax.numpy as jnp
from jax import lax
from jax.experimental import pallas as pl
from jax.experimental.pallas import tpu as pltpu
from jax.experimental.pallas import tpu_sc as plsc

N = 10000
E = 320000
D = 128

NC = 2
NS = 16
NW = NC * NS
EPW = E // NW
C = 40
G = EPW // C
NCH = E // C
NPAD = 10240
RPT = NPAD // NS
LPR = D // 16


def _sc_edge_body(idx_hbm, x_hbm, ef_hbm, zeros_hbm, out_hbm,
                  idq0, idq1, idq2, idq3,
                  efv0, efv1, xv0, xv1, mv0, mv1,
                  sef0, sef1, sx0, sx1, ssc0, ssc1, ssi0, ssi1, ssi2, ssi3,
                  agg_sh):
    idq = (idq0, idq1, idq2, idq3)
    efv = (efv0, efv1)
    xv = (xv0, xv1)
    mv = (mv0, mv1)
    sef = (sef0, sef1)
    sx = (sx0, sx1)
    ssc = (ssc0, ssc1)
    ssi = (ssi0, ssi1, ssi2, ssi3)

    c = lax.axis_index("c")
    s = lax.axis_index("s")
    wid = c * NS + s
    ibase = wid * EPW
    cbase = wid * G

    pltpu.sync_copy(zeros_hbm, agg_sh.at[pl.ds(s * RPT, RPT)])

    plsc.subcore_barrier()

    def start_idq(g, q):
        ebase = pl.multiple_of(ibase + g * C, 8)
        pltpu.async_copy(idx_hbm.at[:, pl.ds(ebase, C)], idq[q], ssi[q])

    def wait_idq(q):
        pltpu.make_async_copy(idx_hbm.at[:, pl.ds(0, C)], idq[q], ssi[q]).wait()

    def start_in(g, b, q):
        ebase = pl.multiple_of(ibase + g * C, 8)
        pltpu.async_copy(ef_hbm.at[pl.ds(ebase, C)], efv[b], sef[b])
        pltpu.async_copy(x_hbm.at[idq[q].at[0]], xv[b], sx[b])

    def wait_in(b, q):
        pltpu.make_async_copy(ef_hbm.at[pl.ds(0, C)], efv[b], sef[b]).wait()
        pltpu.make_async_copy(x_hbm.at[idq[q].at[0]], xv[b], sx[b]).wait()

    def compute(b):
        @plsc.parallel_loop(0, C, 1, unroll=4)
        def _(r):
            for j in range(LPR):
                sl = pl.ds(j * 16, 16)
                mv[b][r, sl] = jnp.maximum(xv[b][r, sl] + efv[b][r, sl], 0.0)

    def start_scatter(b, q):
        pltpu.async_copy(mv[b], agg_sh.at[idq[q].at[1]], ssc[b], add=True)

    def wait_scatter(b, q):
        pltpu.make_async_copy(mv[b], agg_sh.at[idq[q].at[1]], ssc[b]).wait()

    def iteration(g, b, q, first):
        def start_next():
            wait_idq((q + 1) % 4)
            start_in(g + 1, b ^ 1, (q + 1) % 4)

        def start_next_idx():
            start_idq(g + 2, (q + 2) % 4)

        if first:
            if g + 1 < G:
                start_next()
        else:
            @pl.when(g + 1 < G)
            def _():
                start_next()
        wait_in(b, q)
        if not first:
            wait_scatter(b, (q + 2) % 4)
        compute(b)
        start_scatter(b, q)
        if first:
            if g + 2 < G:
                start_next_idx()
        else:
            @pl.when(g + 2 < G)
            def _():
                start_next_idx()

    start_idq(0, 0)
    wait_idq(0)
    start_in(0, 0, 0)
    start_idq(1, 1)

    iteration(0, 0, 0, True)
    iteration(1, 1, 1, True)

    def quad(i, carry):
        g = 2 + 4 * i
        iteration(g, 0, 2, False)
        iteration(g + 1, 1, 3, False)
        iteration(g + 2, 0, 0, False)
        iteration(g + 3, 1, 1, False)
        return carry

    lax.fori_loop(0, (G - 2) // 4, quad, 0)

    wait_scatter(0, (G - 2) % 4)
    wait_scatter(1, (G - 1) % 4)

    plsc.subcore_barrier()

    rbase = s * RPT
    pltpu.sync_copy(agg_sh.at[pl.ds(rbase, RPT)],
                    out_hbm.at[c, pl.ds(rbase, RPT)])


@jax.jit
def _sc_edge_phase(idx2, x, ef, zeros):
    mesh = plsc.VectorSubcoreMesh(core_axis_name="c", subcore_axis_name="s")
    k = pl.kernel(
        _sc_edge_body,
        out_type=jax.ShapeDtypeStruct((NC, NPAD, D), jnp.float32),
        mesh=mesh,
        compiler_params=pltpu.CompilerParams(use_tc_tiling_on_sc=False),
        scratch_types=[
            pltpu.VMEM((2, C), jnp.int32),
            pltpu.VMEM((2, C), jnp.int32),
            pltpu.VMEM((2, C), jnp.int32),
            pltpu.VMEM((2, C), jnp.int32),
            pltpu.VMEM((C, D), jnp.float32),
            pltpu.VMEM((C, D), jnp.float32),
            pltpu.VMEM((C, D), jnp.float32),
            pltpu.VMEM((C, D), jnp.float32),
            pltpu.VMEM((C, D), jnp.float32),
            pltpu.VMEM((C, D), jnp.float32),
            pltpu.SemaphoreType.DMA,
            pltpu.SemaphoreType.DMA,
            pltpu.SemaphoreType.DMA,
            pltpu.SemaphoreType.DMA,
            pltpu.SemaphoreType.DMA,
            pltpu.SemaphoreType.DMA,
            pltpu.SemaphoreType.DMA,
            pltpu.SemaphoreType.DMA,
            pltpu.SemaphoreType.DMA,
            pltpu.SemaphoreType.DMA,
            pltpu.VMEM_SHARED((NPAD, D), jnp.float32),
        ],
    )
    return k(idx2, x, ef, zeros)


def _mlp_body(x_ref, p_ref, w1_ref, b1_ref, w2_ref, b2_ref, o_ref):
    h = x_ref[...] + p_ref[0] + p_ref[1]
    h = jnp.maximum(
        jnp.dot(h, w1_ref[...], preferred_element_type=jnp.float32) + b1_ref[...],
        0.0)
    o_ref[...] = (
        jnp.dot(h, w2_ref[...], preferred_element_type=jnp.float32) + b2_ref[...])


BR = 400


@jax.jit
def _mlp_phase(x, p, W1, b1, W2, b2):
    grid = (N // BR,)
    return pl.pallas_call(
        _mlp_body,
        grid=grid,
        in_specs=[
            pl.BlockSpec((BR, D), lambda i: (i, 0)),
            pl.BlockSpec((NC, BR, D), lambda i: (0, i, 0)),
            pl.BlockSpec((D, D), lambda i: (0, 0)),
            pl.BlockSpec((1, D), lambda i: (0, 0)),
            pl.BlockSpec((D, D), lambda i: (0, 0)),
            pl.BlockSpec((1, D), lambda i: (0, 0)),
        ],
        out_specs=pl.BlockSpec((BR, D), lambda i: (i, 0)),
        out_shape=jax.ShapeDtypeStruct((N, D), jnp.float32),
    )(x, p, W1, b1.reshape(1, D), W2, b2.reshape(1, D))


def kernel(x, edge_index, identifiers, degrees, edge_features, W1, b1, W2, b2):
    zeros = jnp.zeros((RPT, D), jnp.float32)
    p = _sc_edge_phase(edge_index, x, edge_features, zeros)
    return _mlp_phase(x, p, W1, b1, W2, b2)


# speedup vs baseline: 7.9798x; 1.2108x over previous
import jax
import jax.numpy as jnp
from jax import lax
from jax.experimental import pallas as pl
from jax.experimental.pallas import tpu as pltpu
from jax.experimental.pallas import tpu_sc as plsc

N = 10000
E = 320000
D = 128

NC = 2
NS = 16
NW = NC * NS
EPW = E // NW
C = 80
G = EPW // C
NPAD = 10112
RPT = NPAD // NS
LPR = D // 16


def _sc_edge_body(idx_hbm, x_hbm, ef_hbm, zeros_hbm, out_hbm,
                  idq0, idq1, idq2, idq3,
                  xv0, xv1, mv0, mv1,
                  sef0, sef1, sx0, sx1, ssc0, ssc1, ssi0, ssi1, ssi2, ssi3,
                  agg_sh):
    idq = (idq0, idq1, idq2, idq3)
    xv = (xv0, xv1)
    mv = (mv0, mv1)
    sef = (sef0, sef1)
    sx = (sx0, sx1)
    ssc = (ssc0, ssc1)
    ssi = (ssi0, ssi1, ssi2, ssi3)

    c = lax.axis_index("c")
    s = lax.axis_index("s")
    wid = c * NS + s
    ibase = wid * EPW

    pltpu.sync_copy(zeros_hbm, agg_sh.at[pl.ds(s * RPT, RPT)])

    plsc.subcore_barrier()

    def start_idq(g, q):
        ebase = pl.multiple_of(ibase + g * C, 8)
        pltpu.async_copy(idx_hbm.at[:, pl.ds(ebase, C)], idq[q], ssi[q])

    def wait_idq(q):
        pltpu.make_async_copy(idx_hbm.at[:, pl.ds(0, C)], idq[q], ssi[q]).wait()

    def start_in(g, b, q):
        ebase = pl.multiple_of(ibase + g * C, 8)
        pltpu.async_copy(ef_hbm.at[pl.ds(ebase, C)], mv[b], sef[b])
        pltpu.async_copy(x_hbm.at[idq[q].at[0]], xv[b], sx[b])

    def wait_in(b, q):
        pltpu.make_async_copy(ef_hbm.at[pl.ds(0, C)], mv[b], sef[b]).wait()
        pltpu.make_async_copy(x_hbm.at[idq[q].at[0]], xv[b], sx[b]).wait()

    def compute(b):
        @plsc.parallel_loop(0, C, 1, unroll=4)
        def _(r):
            for j in range(LPR):
                sl = pl.ds(j * 16, 16)
                mv[b][r, sl] = jnp.maximum(mv[b][r, sl] + xv[b][r, sl], 0.0)

    def start_scatter(b, q):
        pltpu.async_copy(mv[b], agg_sh.at[idq[q].at[1]], ssc[b], add=True)

    def wait_scatter(b, q):
        pltpu.make_async_copy(mv[b], agg_sh.at[idq[q].at[1]], ssc[b]).wait()

    def iteration(g, b, q, first):
        def start_next():
            wait_idq((q + 1) % 4)
            if not first:
                wait_scatter(b ^ 1, (q + 3) % 4)
            start_in(g + 1, b ^ 1, (q + 1) % 4)

        def start_next_idx():
            start_idq(g + 2, (q + 2) % 4)

        if first:
            if g + 1 < G:
                start_next()
        else:
            @pl.when(g + 1 < G)
            def _():
                start_next()
        wait_in(b, q)
        compute(b)
        start_scatter(b, q)
        if first:
            if g + 2 < G:
                start_next_idx()
        else:
            @pl.when(g + 2 < G)
            def _():
                start_next_idx()

    start_idq(0, 0)
    wait_idq(0)
    start_in(0, 0, 0)
    start_idq(1, 1)

    iteration(0, 0, 0, True)

    def quad(i, carry):
        g = 1 + 4 * i
        iteration(g, 1, 1, False)
        iteration(g + 1, 0, 2, False)
        iteration(g + 2, 1, 3, False)
        iteration(g + 3, 0, 0, False)
        return carry

    lax.fori_loop(0, (G - 1) // 4, quad, 0)

    wait_scatter(1, (G - 2) % 4)
    wait_scatter(0, (G - 1) % 4)

    plsc.subcore_barrier()

    rbase = s * RPT
    pltpu.sync_copy(agg_sh.at[pl.ds(rbase, RPT)],
                    out_hbm.at[c, pl.ds(rbase, RPT)])


@jax.jit
def _sc_edge_phase(idx2, x, ef, zeros):
    mesh = plsc.VectorSubcoreMesh(core_axis_name="c", subcore_axis_name="s")
    k = pl.kernel(
        _sc_edge_body,
        out_type=jax.ShapeDtypeStruct((NC, NPAD, D), jnp.float32),
        mesh=mesh,
        compiler_params=pltpu.CompilerParams(use_tc_tiling_on_sc=False),
        scratch_types=[
            pltpu.VMEM((2, C), jnp.int32),
            pltpu.VMEM((2, C), jnp.int32),
            pltpu.VMEM((2, C), jnp.int32),
            pltpu.VMEM((2, C), jnp.int32),
            pltpu.VMEM((C, D), jnp.float32),
            pltpu.VMEM((C, D), jnp.float32),
            pltpu.VMEM((C, D), jnp.float32),
            pltpu.VMEM((C, D), jnp.float32),
            pltpu.SemaphoreType.DMA,
            pltpu.SemaphoreType.DMA,
            pltpu.SemaphoreType.DMA,
            pltpu.SemaphoreType.DMA,
            pltpu.SemaphoreType.DMA,
            pltpu.SemaphoreType.DMA,
            pltpu.SemaphoreType.DMA,
            pltpu.SemaphoreType.DMA,
            pltpu.SemaphoreType.DMA,
            pltpu.SemaphoreType.DMA,
            pltpu.VMEM_SHARED((NPAD, D), jnp.float32),
        ],
    )
    return k(idx2, x, ef, zeros)


def _mlp_body(x_ref, p_ref, w1_ref, b1_ref, w2_ref, b2_ref, o_ref):
    h = x_ref[...] + p_ref[0] + p_ref[1]
    h = jnp.maximum(
        jnp.dot(h, w1_ref[...], preferred_element_type=jnp.float32) + b1_ref[...],
        0.0)
    o_ref[...] = (
        jnp.dot(h, w2_ref[...], preferred_element_type=jnp.float32) + b2_ref[...])


BR = 400


@jax.jit
def _mlp_phase(x, p, W1, b1, W2, b2):
    grid = (N // BR,)
    return pl.pallas_call(
        _mlp_body,
        grid=grid,
        in_specs=[
            pl.BlockSpec((BR, D), lambda i: (i, 0)),
            pl.BlockSpec((NC, BR, D), lambda i: (0, i, 0)),
            pl.BlockSpec((D, D), lambda i: (0, 0)),
            pl.BlockSpec((1, D), lambda i: (0, 0)),
            pl.BlockSpec((D, D), lambda i: (0, 0)),
            pl.BlockSpec((1, D), lambda i: (0, 0)),
        ],
        out_specs=pl.BlockSpec((BR, D), lambda i: (i, 0)),
        out_shape=jax.ShapeDtypeStruct((N, D), jnp.float32),
    )(x, p, W1, b1.reshape(1, D), W2, b2.reshape(1, D))


def kernel(x, edge_index, identifiers, degrees, edge_features, W1, b1, W2, b2):
    zeros = jnp.zeros((RPT, D), jnp.float32)
    p = _sc_edge_phase(edge_index, x, edge_features, zeros)
    return _mlp_phase(x, p, W1, b1, W2, b2)


# speedup vs baseline: 8.1938x; 1.0268x over previous
import jax
import jax.numpy as jnp
from jax import lax
from jax.experimental import pallas as pl
from jax.experimental.pallas import tpu as pltpu
from jax.experimental.pallas import tpu_sc as plsc

N = 10000
E = 320000
D = 128

NC = 2
NS = 16
NW = NC * NS
EPW = E // NW
C = 80
G = EPW // C
S = 4
NSUP = (G - 1) // S
NPAD = 10112
RPT = NPAD // NS
LPR = D // 16


def _sc_edge_body(idx_hbm, x_hbm, ef_hbm, zeros_hbm, out_hbm,
                  idq0, big0, big1, xv0, xv1, mv0, mv1,
                  sq0, sbig0, sbig1, sef0, sef1, sx0, sx1, ssc0, ssc1,
                  agg_sh):
    big = (big0, big1)
    xv = (xv0, xv1)
    mv = (mv0, mv1)
    sbig = (sbig0, sbig1)
    sef = (sef0, sef1)
    sx = (sx0, sx1)
    ssc = (ssc0, ssc1)

    c = lax.axis_index("c")
    s = lax.axis_index("s")
    wid = c * NS + s
    ibase = wid * EPW

    pltpu.sync_copy(zeros_hbm, agg_sh.at[pl.ds(s * RPT, RPT)])

    plsc.subcore_barrier()

    def start_big(j, B):
        ebase = pl.multiple_of(ibase + (1 + S * j) * C, 8)
        pltpu.async_copy(idx_hbm.at[:, pl.ds(ebase, S * C)], big[B], sbig[B])

    def wait_big(B):
        pltpu.make_async_copy(
            idx_hbm.at[:, pl.ds(0, S * C)], big[B], sbig[B]).wait()

    def start_in(gofs, j, b, sidx_ref):
        ebase = pl.multiple_of(ibase + (1 + S * j + gofs - 1) * C, 8)
        pltpu.async_copy(ef_hbm.at[pl.ds(ebase, C)], mv[b], sef[b])
        pltpu.async_copy(x_hbm.at[sidx_ref], xv[b], sx[b])

    def wait_in(b):
        pltpu.make_async_copy(ef_hbm.at[pl.ds(0, C)], mv[b], sef[b]).wait()
        pltpu.make_async_copy(x_hbm.at[idq0.at[0]], xv[b], sx[b]).wait()

    def compute(b):
        @plsc.parallel_loop(0, C, 1, unroll=4)
        def _(r):
            for j in range(LPR):
                sl = pl.ds(j * 16, 16)
                mv[b][r, sl] = jnp.maximum(mv[b][r, sl] + xv[b][r, sl], 0.0)

    def start_scatter(b, didx_ref):
        pltpu.async_copy(mv[b], agg_sh.at[didx_ref], ssc[b], add=True)

    def wait_scatter(b):
        pltpu.make_async_copy(mv[b], agg_sh.at[idq0.at[1]], ssc[b]).wait()

    pltpu.async_copy(idx_hbm.at[:, pl.ds(pl.multiple_of(ibase, 8), C)],
                     idq0, sq0)
    start_big(0, 0)
    pltpu.make_async_copy(idx_hbm.at[:, pl.ds(0, C)], idq0, sq0).wait()
    pltpu.async_copy(ef_hbm.at[pl.ds(pl.multiple_of(ibase, 8), C)],
                     mv[0], sef[0])
    pltpu.async_copy(x_hbm.at[idq0.at[0]], xv[0], sx[0])

    wait_big(0)
    start_in(1, 0, 1, big[0].at[0, pl.ds(0, C)])
    wait_in(0)
    compute(0)
    start_scatter(0, idq0.at[1])

    def super_chunk(j, carry):
        B = j % 2

        for k in range(S):
            b = (1 + k) % 2
            wait_scatter(b ^ 1)
            if k < S - 1:
                guard0 = j % 2 == 0
                guard1 = j % 2 == 1
            else:
                guard0 = (j % 2 == 0) & (j < NSUP - 1)
                guard1 = (j % 2 == 1) & (j < NSUP - 1)

            @pl.when(guard0)
            def _():
                if k == S - 1:
                    wait_big(1)
                    start_in(k + 2, j, b ^ 1, big[1].at[0, pl.ds(0, C)])
                else:
                    start_in(k + 2, j, b ^ 1,
                             big[0].at[0, pl.ds((k + 1) * C, C)])

            @pl.when(guard1)
            def _():
                if k == S - 1:
                    wait_big(0)
                    start_in(k + 2, j, b ^ 1, big[0].at[0, pl.ds(0, C)])
                else:
                    start_in(k + 2, j, b ^ 1,
                             big[1].at[0, pl.ds((k + 1) * C, C)])
            wait_in(b)
            compute(b)
            @pl.when(j % 2 == 0)
            def _():
                start_scatter(b, big[0].at[1, pl.ds(k * C, C)])

            @pl.when(j % 2 == 1)
            def _():
                start_scatter(b, big[1].at[1, pl.ds(k * C, C)])
            if k == 1:
                @pl.when(j < NSUP - 1)
                def _():
                    @pl.when(j % 2 == 0)
                    def _():
                        start_big(j + 1, 1)

                    @pl.when(j % 2 == 1)
                    def _():
                        start_big(j + 1, 0)
        return carry

    lax.fori_loop(0, NSUP, super_chunk, 0)

    wait_scatter(0)

    plsc.subcore_barrier()

    rbase = s * RPT
    pltpu.sync_copy(agg_sh.at[pl.ds(rbase, RPT)],
                    out_hbm.at[c, pl.ds(rbase, RPT)])


@jax.jit
def _sc_edge_phase(idx2, x, ef, zeros):
    mesh = plsc.VectorSubcoreMesh(core_axis_name="c", subcore_axis_name="s")
    k = pl.kernel(
        _sc_edge_body,
        out_type=jax.ShapeDtypeStruct((NC, NPAD, D), jnp.float32),
        mesh=mesh,
        compiler_params=pltpu.CompilerParams(use_tc_tiling_on_sc=False),
        scratch_types=[
            pltpu.VMEM((2, C), jnp.int32),
            pltpu.VMEM((2, S * C), jnp.int32),
            pltpu.VMEM((2, S * C), jnp.int32),
            pltpu.VMEM((C, D), jnp.float32),
            pltpu.VMEM((C, D), jnp.float32),
            pltpu.VMEM((C, D), jnp.float32),
            pltpu.VMEM((C, D), jnp.float32),
            pltpu.SemaphoreType.DMA,
            pltpu.SemaphoreType.DMA,
            pltpu.SemaphoreType.DMA,
            pltpu.SemaphoreType.DMA,
            pltpu.SemaphoreType.DMA,
            pltpu.SemaphoreType.DMA,
            pltpu.SemaphoreType.DMA,
            pltpu.SemaphoreType.DMA,
            pltpu.SemaphoreType.DMA,
            pltpu.VMEM_SHARED((NPAD, D), jnp.float32),
        ],
    )
    return k(idx2, x, ef, zeros)


def _mlp_body(x_ref, p_ref, w1_ref, b1_ref, w2_ref, b2_ref, o_ref):
    h = x_ref[...] + p_ref[0] + p_ref[1]
    h = jnp.maximum(
        jnp.dot(h, w1_ref[...], preferred_element_type=jnp.float32) + b1_ref[...],
        0.0)
    o_ref[...] = (
        jnp.dot(h, w2_ref[...], preferred_element_type=jnp.float32) + b2_ref[...])


BR = 400


@jax.jit
def _mlp_phase(x, p, W1, b1, W2, b2):
    grid = (N // BR,)
    return pl.pallas_call(
        _mlp_body,
        grid=grid,
        in_specs=[
            pl.BlockSpec((BR, D), lambda i: (i, 0)),
            pl.BlockSpec((NC, BR, D), lambda i: (0, i, 0)),
            pl.BlockSpec((D, D), lambda i: (0, 0)),
            pl.BlockSpec((1, D), lambda i: (0, 0)),
            pl.BlockSpec((D, D), lambda i: (0, 0)),
            pl.BlockSpec((1, D), lambda i: (0, 0)),
        ],
        out_specs=pl.BlockSpec((BR, D), lambda i: (i, 0)),
        out_shape=jax.ShapeDtypeStruct((N, D), jnp.float32),
    )(x, p, W1, b1.reshape(1, D), W2, b2.reshape(1, D))


def kernel(x, edge_index, identifiers, degrees, edge_features, W1, b1, W2, b2):
    zeros = jnp.zeros((RPT, D), jnp.float32)
    p = _sc_edge_phase(edge_index, x, edge_features, zeros)
    return _mlp_phase(x, p, W1, b1, W2, b2)


# speedup vs baseline: 8.7288x; 1.0653x over previous
import jax
import jax.numpy as jnp
from jax import lax
from jax.experimental import pallas as pl
from jax.experimental.pallas import tpu as pltpu
from jax.experimental.pallas import tpu_sc as plsc

N = 10000
E = 320000
D = 128

NC = 2
NS = 16
NW = NC * NS
EPW = E // NW
C = 80
G = EPW // C
S = 4
NSUP = (G - 1) // S
NPAD = 10112
RPT = NPAD // NS
LPR = D // 16


def _sc_edge_body(idx_hbm, x_hbm, ef_hbm, out_hbm,
                  idq0, big0, big1, xv0, xv1, mv0, mv1,
                  sq0, sbig0, sbig1, sef0, sef1, sx0, sx1, ssc0, ssc1,
                  agg_sh):
    big = (big0, big1)
    xv = (xv0, xv1)
    mv = (mv0, mv1)
    sbig = (sbig0, sbig1)
    sef = (sef0, sef1)
    sx = (sx0, sx1)
    ssc = (ssc0, ssc1)

    c = lax.axis_index("c")
    s = lax.axis_index("s")
    wid = c * NS + s
    ibase = wid * EPW

    @plsc.parallel_loop(0, C, 1, unroll=4)
    def _(r):
        for j in range(LPR):
            mv0[r, pl.ds(j * 16, 16)] = jnp.zeros((16,), jnp.float32)

    zbase = s * RPT
    for t in range(RPT // C):
        pltpu.sync_copy(mv0, agg_sh.at[pl.ds(zbase + t * C, C)])
    rem = RPT % C
    if rem:
        pltpu.sync_copy(mv0.at[pl.ds(0, rem)],
                        agg_sh.at[pl.ds(zbase + (RPT // C) * C, rem)])

    plsc.subcore_barrier()

    def start_big(j, B):
        ebase = pl.multiple_of(ibase + (1 + S * j) * C, 8)
        pltpu.async_copy(idx_hbm.at[:, pl.ds(ebase, S * C)], big[B], sbig[B])

    def wait_big(B):
        pltpu.make_async_copy(
            idx_hbm.at[:, pl.ds(0, S * C)], big[B], sbig[B]).wait()

    def start_in(gofs, j, b, sidx_ref):
        ebase = pl.multiple_of(ibase + (1 + S * j + gofs - 1) * C, 8)
        pltpu.async_copy(ef_hbm.at[pl.ds(ebase, C)], mv[b], sef[b])
        pltpu.async_copy(x_hbm.at[sidx_ref], xv[b], sx[b])

    def wait_in(b):
        pltpu.make_async_copy(ef_hbm.at[pl.ds(0, C)], mv[b], sef[b]).wait()
        pltpu.make_async_copy(x_hbm.at[idq0.at[0]], xv[b], sx[b]).wait()

    def compute(b):
        @plsc.parallel_loop(0, C, 1, unroll=4)
        def _(r):
            for j in range(LPR):
                sl = pl.ds(j * 16, 16)
                mv[b][r, sl] = jnp.maximum(mv[b][r, sl] + xv[b][r, sl], 0.0)

    def start_scatter(b, didx_ref):
        pltpu.async_copy(mv[b], agg_sh.at[didx_ref], ssc[b], add=True)

    def wait_scatter(b):
        pltpu.make_async_copy(mv[b], agg_sh.at[idq0.at[1]], ssc[b]).wait()

    pltpu.async_copy(idx_hbm.at[:, pl.ds(pl.multiple_of(ibase, 8), C)],
                     idq0, sq0)
    start_big(0, 0)
    pltpu.make_async_copy(idx_hbm.at[:, pl.ds(0, C)], idq0, sq0).wait()
    pltpu.async_copy(ef_hbm.at[pl.ds(pl.multiple_of(ibase, 8), C)],
                     mv[0], sef[0])
    pltpu.async_copy(x_hbm.at[idq0.at[0]], xv[0], sx[0])

    wait_big(0)
    start_in(1, 0, 1, big[0].at[0, pl.ds(0, C)])
    wait_in(0)
    compute(0)
    start_scatter(0, idq0.at[1])

    def super_chunk(j, carry):
        B = j % 2

        for k in range(S):
            b = (1 + k) % 2
            wait_scatter(b ^ 1)
            if k < S - 1:
                guard0 = j % 2 == 0
                guard1 = j % 2 == 1
            else:
                guard0 = (j % 2 == 0) & (j < NSUP - 1)
                guard1 = (j % 2 == 1) & (j < NSUP - 1)

            @pl.when(guard0)
            def _():
                if k == S - 1:
                    wait_big(1)
                    start_in(k + 2, j, b ^ 1, big[1].at[0, pl.ds(0, C)])
                else:
                    start_in(k + 2, j, b ^ 1,
                             big[0].at[0, pl.ds((k + 1) * C, C)])

            @pl.when(guard1)
            def _():
                if k == S - 1:
                    wait_big(0)
                    start_in(k + 2, j, b ^ 1, big[0].at[0, pl.ds(0, C)])
                else:
                    start_in(k + 2, j, b ^ 1,
                             big[1].at[0, pl.ds((k + 1) * C, C)])
            wait_in(b)
            compute(b)
            @pl.when(j % 2 == 0)
            def _():
                start_scatter(b, big[0].at[1, pl.ds(k * C, C)])

            @pl.when(j % 2 == 1)
            def _():
                start_scatter(b, big[1].at[1, pl.ds(k * C, C)])
            if k == 1:
                @pl.when(j < NSUP - 1)
                def _():
                    @pl.when(j % 2 == 0)
                    def _():
                        start_big(j + 1, 1)

                    @pl.when(j % 2 == 1)
                    def _():
                        start_big(j + 1, 0)
        return carry

    lax.fori_loop(0, NSUP, super_chunk, 0)

    wait_scatter(0)

    plsc.subcore_barrier()

    rbase = s * RPT
    pltpu.sync_copy(agg_sh.at[pl.ds(rbase, RPT)],
                    out_hbm.at[c, pl.ds(rbase, RPT)])


@jax.jit
def _sc_edge_phase(idx2, x, ef):
    mesh = plsc.VectorSubcoreMesh(core_axis_name="c", subcore_axis_name="s")
    k = pl.kernel(
        _sc_edge_body,
        out_type=jax.ShapeDtypeStruct((NC, NPAD, D), jnp.float32),
        mesh=mesh,
        compiler_params=pltpu.CompilerParams(use_tc_tiling_on_sc=False),
        scratch_types=[
            pltpu.VMEM((2, C), jnp.int32),
            pltpu.VMEM((2, S * C), jnp.int32),
            pltpu.VMEM((2, S * C), jnp.int32),
            pltpu.VMEM((C, D), jnp.float32),
            pltpu.VMEM((C, D), jnp.float32),
            pltpu.VMEM((C, D), jnp.float32),
            pltpu.VMEM((C, D), jnp.float32),
            pltpu.SemaphoreType.DMA,
            pltpu.SemaphoreType.DMA,
            pltpu.SemaphoreType.DMA,
            pltpu.SemaphoreType.DMA,
            pltpu.SemaphoreType.DMA,
            pltpu.SemaphoreType.DMA,
            pltpu.SemaphoreType.DMA,
            pltpu.SemaphoreType.DMA,
            pltpu.SemaphoreType.DMA,
            pltpu.VMEM_SHARED((NPAD, D), jnp.float32),
        ],
    )
    return k(idx2, x, ef)


def _mlp_body(x_ref, p_ref, w1_ref, b1_ref, w2_ref, b2_ref, o_ref):
    h = x_ref[...] + p_ref[0] + p_ref[1]
    h = jnp.maximum(
        jnp.dot(h, w1_ref[...], preferred_element_type=jnp.float32) + b1_ref[...],
        0.0)
    o_ref[...] = (
        jnp.dot(h, w2_ref[...], preferred_element_type=jnp.float32) + b2_ref[...])


BR = 2000


@jax.jit
def _mlp_phase(x, p, W1, b1, W2, b2):
    grid = (N // BR,)
    return pl.pallas_call(
        _mlp_body,
        grid=grid,
        in_specs=[
            pl.BlockSpec((BR, D), lambda i: (i, 0)),
            pl.BlockSpec((NC, BR, D), lambda i: (0, i, 0)),
            pl.BlockSpec((D, D), lambda i: (0, 0)),
            pl.BlockSpec((1, D), lambda i: (0, 0)),
            pl.BlockSpec((D, D), lambda i: (0, 0)),
            pl.BlockSpec((1, D), lambda i: (0, 0)),
        ],
        out_specs=pl.BlockSpec((BR, D), lambda i: (i, 0)),
        out_shape=jax.ShapeDtypeStruct((N, D), jnp.float32),
    )(x, p, W1, b1.reshape(1, D), W2, b2.reshape(1, D))


def kernel(x, edge_index, identifiers, degrees, edge_features, W1, b1, W2, b2):
    p = _sc_edge_phase(edge_index, x, edge_features)
    return _mlp_phase(x, p, W1, b1, W2, b2)
